# baseline (speedup 1.0000x reference)
import functools

import jax
import jax.numpy as jnp
from jax import lax
from jax.experimental import pallas as pl
from jax.experimental.pallas import tpu as pltpu
from jax.experimental.pallas import tpu_sc as plsc

N = 10000
E = 320000
HID = 128
DCAT = 64
NCLS = 2

NC = 2
NS = 16
NW = NC * NS
CHUNK = 128
NCHT = 158
NCH0 = 105
NCH1 = NCHT - NCH0
NCH_MAX = max(NCH0, NCH1)
NPAD = 10240
RPT = NPAD // NS

BN = 5000
GRID = N // BN


def _segsum_body(m_hbm, src_hbm, dst_hbm, zeros_hbm, out_hbm,
                 src_v, didx, rows0, rows1, agg_sh,
                 semr0, semr1, semd0, semd1):
    c = lax.axis_index("c")
    s = lax.axis_index("s")
    nch = jnp.where(c == 0, NCH0, NCH1)
    npairs = (nch - 1) // 2

    pltpu.sync_copy(src_hbm.at[s].at[c], src_v)

    @pl.when(s == 0)
    def _():
        pltpu.sync_copy(zeros_hbm, agg_sh)

    plsc.subcore_barrier()

    dst_t = dst_hbm.at[s].at[c]

    pltpu.async_copy(dst_t.at[pl.ds(0, 1)], didx.at[pl.ds(0, 1)], semd0)
    pltpu.async_copy(m_hbm.at[src_v.at[0]], rows0, semr0)
    pltpu.async_copy(dst_t.at[pl.ds(1, 1)], didx.at[pl.ds(1, 1)], semd1)
    pltpu.async_copy(m_hbm.at[src_v.at[1]], rows1, semr1)

    @pl.loop(0, npairs)
    def _(p):
        j0 = 2 * p

        pltpu.make_async_copy(m_hbm.at[src_v.at[j0]], rows0, semr0).wait()
        pltpu.make_async_copy(dst_t.at[pl.ds(j0, 1)], didx.at[pl.ds(0, 1)], semd0).wait()
        pltpu.sync_copy(rows0, agg_sh.at[didx.at[0]], add=True)
        pltpu.async_copy(dst_t.at[pl.ds(j0 + 2, 1)], didx.at[pl.ds(0, 1)], semd0)
        pltpu.async_copy(m_hbm.at[src_v.at[j0 + 2]], rows0, semr0)

        pltpu.make_async_copy(m_hbm.at[src_v.at[j0 + 1]], rows1, semr1).wait()
        pltpu.make_async_copy(dst_t.at[pl.ds(j0 + 1, 1)], didx.at[pl.ds(1, 1)], semd1).wait()
        pltpu.sync_copy(rows1, agg_sh.at[didx.at[1]], add=True)

        @pl.when(p < npairs - 1)
        def _():
            pltpu.async_copy(dst_t.at[pl.ds(j0 + 3, 1)], didx.at[pl.ds(1, 1)], semd1)
            pltpu.async_copy(m_hbm.at[src_v.at[j0 + 3]], rows1, semr1)

    pltpu.make_async_copy(m_hbm.at[src_v.at[nch - 1]], rows0, semr0).wait()
    pltpu.make_async_copy(dst_t.at[pl.ds(nch - 1, 1)], didx.at[pl.ds(0, 1)], semd0).wait()
    pltpu.sync_copy(rows0, agg_sh.at[didx.at[0]], add=True)

    plsc.subcore_barrier()

    pltpu.sync_copy(agg_sh.at[pl.ds(s * RPT, RPT)],
                    out_hbm.at[c].at[pl.ds(s * RPT, RPT)])


_segsum = functools.partial(
    pl.kernel,
    out_type=jax.ShapeDtypeStruct((NC, NPAD, HID), jnp.float32),
    mesh=plsc.VectorSubcoreMesh(core_axis_name="c", subcore_axis_name="s"),
    scratch_types=[
        pltpu.VMEM((NCH_MAX, CHUNK), jnp.int32),
        pltpu.VMEM((2, CHUNK), jnp.int32),
        pltpu.VMEM((CHUNK, HID), jnp.float32),
        pltpu.VMEM((CHUNK, HID), jnp.float32),
        pltpu.VMEM_SHARED((NPAD, HID), jnp.float32),
        pltpu.SemaphoreType.DMA,
        pltpu.SemaphoreType.DMA,
        pltpu.SemaphoreType.DMA,
        pltpu.SemaphoreType.DMA,
    ],
)(_segsum_body)


def _enc_body(xc_ref, xs_ref, wpt_ref, bp_ref, wst_ref, bs_ref, wg_ref,
              h_ref, m_ref):
    xc = jnp.dot(xc_ref[...], wpt_ref[...],
                 preferred_element_type=jnp.float32) + bp_ref[...]
    xs = jnp.dot(xs_ref[...], wst_ref[...],
                 preferred_element_type=jnp.float32) + bs_ref[...]
    h = jnp.concatenate([xc, xs], axis=1)
    h_ref[...] = h
    m_ref[...] = jnp.dot(h, wg_ref[...], preferred_element_type=jnp.float32)


def _gru(agg, h, wih, whh, bih, bhh):
    gi = jnp.dot(agg, wih, preferred_element_type=jnp.float32) + bih
    gh = jnp.dot(h, whh, preferred_element_type=jnp.float32) + bhh
    r = jax.nn.sigmoid(gi[:, :HID] + gh[:, :HID])
    z = jax.nn.sigmoid(gi[:, HID:2 * HID] + gh[:, HID:2 * HID])
    n = jnp.tanh(gi[:, 2 * HID:] + r * gh[:, 2 * HID:])
    return (1.0 - z) * n + z * h


def _gru_m_body(agg2_ref, h_ref, wih_ref, whh_ref, bih_ref, bhh_ref, wg_ref,
                hn_ref, m_ref):
    agg = agg2_ref[0] + agg2_ref[1]
    hn = _gru(agg, h_ref[...], wih_ref[...], whh_ref[...],
              bih_ref[...], bhh_ref[...])
    hn_ref[...] = hn
    m_ref[...] = jnp.dot(hn, wg_ref[...], preferred_element_type=jnp.float32)


def _gru_out_body(agg2_ref, h_ref, wih_ref, whh_ref, bih_ref, bhh_ref,
                  wout_ref, bout_ref, out_ref):
    agg = agg2_ref[0] + agg2_ref[1]
    hn = _gru(agg, h_ref[...], wih_ref[...], whh_ref[...],
              bih_ref[...], bhh_ref[...])
    hr = jnp.maximum(hn, 0.0)
    out_ref[...] = jnp.dot(hr, wout_ref[...],
                           preferred_element_type=jnp.float32) + bout_ref[...]


def _row_spec(last):
    return pl.BlockSpec((BN, last), lambda i: (i, 0))


def _full_spec(shape):
    return pl.BlockSpec(shape, lambda i: tuple(0 for _ in shape))


def _agg_spec():
    return pl.BlockSpec((NC, BN, HID), lambda i: (0, i, 0))


def kernel(x_content, x_style, edge_index, edge_type, W_post, b_post,
           W_style, b_style, Wg, W_ih, W_hh, b_ih, b_hh, W_out, b_out):
    del edge_type

    pad = NS * NCHT * CHUNK - E
    pad_dst = N + jnp.arange(pad, dtype=jnp.int32) % (NPAD - N)
    src = jnp.concatenate([edge_index[0], jnp.zeros((pad,), jnp.int32)])
    dst = jnp.concatenate([edge_index[1], pad_dst])
    def _split(a):
        a = a.reshape(NS, NCHT, CHUNK)
        a0 = jnp.pad(a[:, :NCH0], ((0, 0), (0, NCH_MAX - NCH0), (0, 0)))
        a1 = jnp.pad(a[:, NCH0:], ((0, 0), (0, NCH_MAX - NCH1), (0, 0)))
        return jnp.stack([a0, a1], axis=1)
    src = _split(src)
    dst = _split(dst)
    zeros = jnp.zeros((NPAD, HID), jnp.float32)

    wpt = W_post.T
    wst = W_style.T
    wih = W_ih.T
    whh = W_hh.T
    wout = W_out.T
    bp = b_post.reshape(1, DCAT)
    bs = b_style.reshape(1, DCAT)
    bih = b_ih.reshape(1, 3 * HID)
    bhh = b_hh.reshape(1, 3 * HID)
    bout = b_out.reshape(1, NCLS)

    h, m = pl.pallas_call(
        _enc_body,
        grid=(GRID,),
        in_specs=[_row_spec(HID), _row_spec(HID),
                  _full_spec((HID, DCAT)), _full_spec((1, DCAT)),
                  _full_spec((HID, DCAT)), _full_spec((1, DCAT)),
                  _full_spec((HID, HID))],
        out_specs=[_row_spec(HID), _row_spec(HID)],
        out_shape=[jax.ShapeDtypeStruct((N, HID), jnp.float32),
                   jax.ShapeDtypeStruct((N, HID), jnp.float32)],
    )(x_content, x_style, wpt, bp, wst, bs, Wg[0])

    agg2 = _segsum(m, src, dst, zeros)
    h, m = pl.pallas_call(
        _gru_m_body,
        grid=(GRID,),
        in_specs=[_agg_spec(), _row_spec(HID),
                  _full_spec((HID, 3 * HID)), _full_spec((HID, 3 * HID)),
                  _full_spec((1, 3 * HID)), _full_spec((1, 3 * HID)),
                  _full_spec((HID, HID))],
        out_specs=[_row_spec(HID), _row_spec(HID)],
        out_shape=[jax.ShapeDtypeStruct((N, HID), jnp.float32),
                   jax.ShapeDtypeStruct((N, HID), jnp.float32)],
    )(agg2, h, wih, whh, bih, bhh, Wg[1])

    agg2 = _segsum(m, src, dst, zeros)
    out = pl.pallas_call(
        _gru_out_body,
        grid=(GRID,),
        in_specs=[_agg_spec(), _row_spec(HID),
                  _full_spec((HID, 3 * HID)), _full_spec((HID, 3 * HID)),
                  _full_spec((1, 3 * HID)), _full_spec((1, 3 * HID)),
                  _full_spec((HID, NCLS)), _full_spec((1, NCLS))],
        out_specs=_row_spec(NCLS),
        out_shape=jax.ShapeDtypeStruct((N, NCLS), jnp.float32),
    )(agg2, h, wih, whh, bih, bhh, wout, bout)

    return out


# speedup vs baseline: 1.0457x; 1.0457x over previous
import functools

import jax
import jax.numpy as jnp
from jax import lax
from jax.experimental import pallas as pl
from jax.experimental.pallas import tpu as pltpu
from jax.experimental.pallas import tpu_sc as plsc

N = 10000
E = 320000
HID = 128
DCAT = 64
NCLS = 2

NC = 2
NS = 16
NW = NC * NS
CHUNK = 128
NCHT = 158
NCH0 = 103
NCH1 = NCHT - NCH0
NCH_MAX = max(NCH0, NCH1)
NPAD = 10240
RPT = NPAD // NS

BN = 5000
GRID = N // BN


def _segsum_body(m_hbm, src_hbm, dst_hbm, zeros_hbm, out_hbm,
                 src_v, didx, rows0, rows1, agg_sh,
                 semr0, semr1, semd0, semd1):
    c = lax.axis_index("c")
    s = lax.axis_index("s")
    nch = jnp.where(c == 0, NCH0, NCH1)
    npairs = (nch - 1) // 2

    pltpu.sync_copy(src_hbm.at[s].at[c], src_v)

    @pl.when(s == 0)
    def _():
        pltpu.sync_copy(zeros_hbm, agg_sh)

    plsc.subcore_barrier()

    dst_t = dst_hbm.at[s].at[c]

    pltpu.async_copy(dst_t.at[pl.ds(0, 1)], didx.at[pl.ds(0, 1)], semd0)
    pltpu.async_copy(m_hbm.at[src_v.at[0]], rows0, semr0)
    pltpu.async_copy(dst_t.at[pl.ds(1, 1)], didx.at[pl.ds(1, 1)], semd1)
    pltpu.async_copy(m_hbm.at[src_v.at[1]], rows1, semr1)

    @pl.loop(0, npairs)
    def _(p):
        j0 = 2 * p

        pltpu.make_async_copy(m_hbm.at[src_v.at[j0]], rows0, semr0).wait()
        pltpu.make_async_copy(dst_t.at[pl.ds(j0, 1)], didx.at[pl.ds(0, 1)], semd0).wait()
        pltpu.sync_copy(rows0, agg_sh.at[didx.at[0]], add=True)
        pltpu.async_copy(dst_t.at[pl.ds(j0 + 2, 1)], didx.at[pl.ds(0, 1)], semd0)
        pltpu.async_copy(m_hbm.at[src_v.at[j0 + 2]], rows0, semr0)

        pltpu.make_async_copy(m_hbm.at[src_v.at[j0 + 1]], rows1, semr1).wait()
        pltpu.make_async_copy(dst_t.at[pl.ds(j0 + 1, 1)], didx.at[pl.ds(1, 1)], semd1).wait()
        pltpu.sync_copy(rows1, agg_sh.at[didx.at[1]], add=True)

        @pl.when(p < npairs - 1)
        def _():
            pltpu.async_copy(dst_t.at[pl.ds(j0 + 3, 1)], didx.at[pl.ds(1, 1)], semd1)
            pltpu.async_copy(m_hbm.at[src_v.at[j0 + 3]], rows1, semr1)

    pltpu.make_async_copy(m_hbm.at[src_v.at[nch - 1]], rows0, semr0).wait()
    pltpu.make_async_copy(dst_t.at[pl.ds(nch - 1, 1)], didx.at[pl.ds(0, 1)], semd0).wait()
    pltpu.sync_copy(rows0, agg_sh.at[didx.at[0]], add=True)

    plsc.subcore_barrier()

    pltpu.sync_copy(agg_sh.at[pl.ds(s * RPT, RPT)],
                    out_hbm.at[c].at[pl.ds(s * RPT, RPT)])


_segsum = functools.partial(
    pl.kernel,
    out_type=jax.ShapeDtypeStruct((NC, NPAD, HID), jnp.float32),
    mesh=plsc.VectorSubcoreMesh(core_axis_name="c", subcore_axis_name="s"),
    scratch_types=[
        pltpu.VMEM((NCH_MAX, CHUNK), jnp.int32),
        pltpu.VMEM((2, CHUNK), jnp.int32),
        pltpu.VMEM((CHUNK, HID), jnp.float32),
        pltpu.VMEM((CHUNK, HID), jnp.float32),
        pltpu.VMEM_SHARED((NPAD, HID), jnp.float32),
        pltpu.SemaphoreType.DMA,
        pltpu.SemaphoreType.DMA,
        pltpu.SemaphoreType.DMA,
        pltpu.SemaphoreType.DMA,
    ],
)(_segsum_body)


def _enc_body(xc_ref, xs_ref, wpt_ref, bp_ref, wst_ref, bs_ref, wg_ref,
              h_ref, m_ref):
    xc = jnp.dot(xc_ref[...], wpt_ref[...],
                 preferred_element_type=jnp.float32) + bp_ref[...]
    xs = jnp.dot(xs_ref[...], wst_ref[...],
                 preferred_element_type=jnp.float32) + bs_ref[...]
    h = jnp.concatenate([xc, xs], axis=1)
    h_ref[...] = h
    m_ref[...] = jnp.dot(h, wg_ref[...], preferred_element_type=jnp.float32)


def _gru(agg, h, wih, whh, bih, bhh):
    gi = jnp.dot(agg, wih, preferred_element_type=jnp.float32) + bih
    gh = jnp.dot(h, whh, preferred_element_type=jnp.float32) + bhh
    r = jax.nn.sigmoid(gi[:, :HID] + gh[:, :HID])
    z = jax.nn.sigmoid(gi[:, HID:2 * HID] + gh[:, HID:2 * HID])
    n = jnp.tanh(gi[:, 2 * HID:] + r * gh[:, 2 * HID:])
    return (1.0 - z) * n + z * h


def _gru_m_body(agg2_ref, h_ref, wih_ref, whh_ref, bih_ref, bhh_ref, wg_ref,
                hn_ref, m_ref):
    agg = agg2_ref[0] + agg2_ref[1]
    hn = _gru(agg, h_ref[...], wih_ref[...], whh_ref[...],
              bih_ref[...], bhh_ref[...])
    hn_ref[...] = hn
    m_ref[...] = jnp.dot(hn, wg_ref[...], preferred_element_type=jnp.float32)


def _gru_out_body(agg2_ref, h_ref, wih_ref, whh_ref, bih_ref, bhh_ref,
                  wout_ref, bout_ref, out_ref):
    agg = agg2_ref[0] + agg2_ref[1]
    hn = _gru(agg, h_ref[...], wih_ref[...], whh_ref[...],
              bih_ref[...], bhh_ref[...])
    hr = jnp.maximum(hn, 0.0)
    out_ref[...] = jnp.dot(hr, wout_ref[...],
                           preferred_element_type=jnp.float32) + bout_ref[...]


def _row_spec(last):
    return pl.BlockSpec((BN, last), lambda i: (i, 0))


def _full_spec(shape):
    return pl.BlockSpec(shape, lambda i: tuple(0 for _ in shape))


def _agg_spec():
    return pl.BlockSpec((NC, BN, HID), lambda i: (0, i, 0))


def kernel(x_content, x_style, edge_index, edge_type, W_post, b_post,
           W_style, b_style, Wg, W_ih, W_hh, b_ih, b_hh, W_out, b_out):
    del edge_type

    pad = NS * NCHT * CHUNK - E
    pad_dst = N + jnp.arange(pad, dtype=jnp.int32) % (NPAD - N)
    src = jnp.concatenate([edge_index[0], jnp.zeros((pad,), jnp.int32)])
    dst = jnp.concatenate([edge_index[1], pad_dst])
    def _split(a):
        a = a.reshape(NS, NCHT, CHUNK)
        a0 = jnp.pad(a[:, :NCH0], ((0, 0), (0, NCH_MAX - NCH0), (0, 0)))
        a1 = jnp.pad(a[:, NCH0:], ((0, 0), (0, NCH_MAX - NCH1), (0, 0)))
        return jnp.stack([a0, a1], axis=1)
    src = _split(src)
    dst = _split(dst)
    zeros = jnp.zeros((NPAD, HID), jnp.float32)

    wpt = W_post.T
    wst = W_style.T
    wih = W_ih.T
    whh = W_hh.T
    wout = W_out.T
    bp = b_post.reshape(1, DCAT)
    bs = b_style.reshape(1, DCAT)
    bih = b_ih.reshape(1, 3 * HID)
    bhh = b_hh.reshape(1, 3 * HID)
    bout = b_out.reshape(1, NCLS)

    h, m = pl.pallas_call(
        _enc_body,
        grid=(GRID,),
        in_specs=[_row_spec(HID), _row_spec(HID),
                  _full_spec((HID, DCAT)), _full_spec((1, DCAT)),
                  _full_spec((HID, DCAT)), _full_spec((1, DCAT)),
                  _full_spec((HID, HID))],
        out_specs=[_row_spec(HID), _row_spec(HID)],
        out_shape=[jax.ShapeDtypeStruct((N, HID), jnp.float32),
                   jax.ShapeDtypeStruct((N, HID), jnp.float32)],
    )(x_content, x_style, wpt, bp, wst, bs, Wg[0])

    agg2 = _segsum(m, src, dst, zeros)
    h, m = pl.pallas_call(
        _gru_m_body,
        grid=(GRID,),
        in_specs=[_agg_spec(), _row_spec(HID),
                  _full_spec((HID, 3 * HID)), _full_spec((HID, 3 * HID)),
                  _full_spec((1, 3 * HID)), _full_spec((1, 3 * HID)),
                  _full_spec((HID, HID))],
        out_specs=[_row_spec(HID), _row_spec(HID)],
        out_shape=[jax.ShapeDtypeStruct((N, HID), jnp.float32),
                   jax.ShapeDtypeStruct((N, HID), jnp.float32)],
    )(agg2, h, wih, whh, bih, bhh, Wg[1])

    agg2 = _segsum(m, src, dst, zeros)
    out = pl.pallas_call(
        _gru_out_body,
        grid=(GRID,),
        in_specs=[_agg_spec(), _row_spec(HID),
                  _full_spec((HID, 3 * HID)), _full_spec((HID, 3 * HID)),
                  _full_spec((1, 3 * HID)), _full_spec((1, 3 * HID)),
                  _full_spec((HID, NCLS)), _full_spec((1, NCLS))],
        out_specs=_row_spec(NCLS),
        out_shape=jax.ShapeDtypeStruct((N, NCLS), jnp.float32),
    )(agg2, h, wih, whh, bih, bhh, wout, bout)

    return out
